# baseline (speedup 1.0000x reference)
import functools

import jax
import jax.numpy as jnp
from jax import lax
from jax.experimental import pallas as pl
from jax.experimental.pallas import tpu as pltpu
from jax.experimental.pallas import tpu_sc as plsc

N = 10000
E = 320000
DH = 128
DOUT = 40
D3 = 128

NC = 2
NS = 16
NT = NC * NS
EPT = E // NT
EC = 80
NCH = EPT // EC
RD = 200
NDCH = N // RD
LK = -(-NDCH // NS)
NDA = N // EC
LKA = -(-NDA // NS)
IB = 32
STAGES = (32, 32, 32, 29)

RB = 2000
GRID = N // RB


def _mesh():
    return plsc.VectorSubcoreMesh(core_axis_name="c", subcore_axis_name="s")


def _deg_body(adj3, zerosd, erows, out, dst_all, ones_v, zbuf, accum):
    cid = lax.axis_index("c")
    sid = lax.axis_index("s")
    tid = cid * NS + sid
    pltpu.sync_copy(erows, ones_v)
    pltpu.sync_copy(zerosd, zbuf)
    for j in range(LKA):
        k = sid + j * NS
        if (j + 1) * NS <= NDA:
            pltpu.sync_copy(zbuf, accum.at[pl.ds(k * EC, EC)])
        else:
            @pl.when(k < NDA)
            def _():
                pltpu.sync_copy(zbuf, accum.at[pl.ds(k * EC, EC)])
    pltpu.sync_copy(adj3.at[1, tid], dst_all)
    plsc.subcore_barrier()

    def body(i, c):
        pltpu.sync_copy(ones_v, accum.at[dst_all.at[i]], add=True)
        return c

    lax.fori_loop(0, NCH, body, 0)
    plsc.subcore_barrier()
    for j in range(LKA):
        k = sid + j * NS
        if (j + 1) * NS <= NDA:
            pltpu.sync_copy(accum.at[pl.ds(k * EC, EC)], zbuf)
            pltpu.sync_copy(zbuf, out.at[cid, pl.ds(k * EC, EC)])
        else:
            @pl.when(k < NDA)
            def _():
                pltpu.sync_copy(accum.at[pl.ds(k * EC, EC)], zbuf)
                pltpu.sync_copy(zbuf, out.at[cid, pl.ds(k * EC, EC)])


_deg_call = pl.kernel(
    _deg_body,
    out_type=jax.ShapeDtypeStruct((NC, N, DH), jnp.float32),
    mesh=_mesh(),
    scratch_types=[
        pltpu.VMEM((NCH, EC), jnp.int32),
        pltpu.VMEM((EC, DH), jnp.float32),
        pltpu.VMEM((EC, DH), jnp.float32),
        pltpu.VMEM_SHARED((N, DH), jnp.float32),
    ],
)


def _make_agg(d):
    def body(hs, adj3, zerosd, out, src_all, dst_all, msg0, msg1, msg2, msg3,
             accum, gsem0, gsem1, gsem2, gsem3, ssem0, ssem1, ssem2, ssem3):
        cid = lax.axis_index("c")
        sid = lax.axis_index("s")
        tid = cid * NS + sid
        pltpu.sync_copy(zerosd, msg0)
        for j in range(LKA):
            k = sid + j * NS
            if (j + 1) * NS <= NDA:
                pltpu.sync_copy(msg0, accum.at[pl.ds(k * EC, EC)])
            else:
                @pl.when(k < NDA)
                def _():
                    pltpu.sync_copy(msg0, accum.at[pl.ds(k * EC, EC)])
        plsc.subcore_barrier()

        msgs = (msg0, msg1, msg2, msg3)
        gsems = (gsem0, gsem1, gsem2, gsem3)
        ssems = (ssem0, ssem1, ssem2, ssem3)

        def wait_gather(idx, b):
            pltpu.make_async_copy(hs.at[src_all.at[idx]], msgs[b],
                                  gsems[b]).wait()

        def wait_scatter(idx, b):
            pltpu.make_async_copy(msgs[b], accum.at[dst_all.at[idx]],
                                  ssems[b]).wait()

        row = 0
        for c in STAGES:
            pltpu.sync_copy(adj3.at[0, tid, pl.ds(row, c)], src_all.at[pl.ds(0, c)])
            pltpu.sync_copy(adj3.at[1, tid, pl.ds(row, c)], dst_all.at[pl.ds(0, c)])
            row += c
            for p in range(3):
                pltpu.async_copy(hs.at[src_all.at[p]], msgs[p], gsems[p])
            for idx in range(4):
                b = idx % 4
                wait_gather(idx, b)
                pltpu.async_copy(msgs[b], accum.at[dst_all.at[idx]], ssems[b],
                                 add=True)
                b3 = (idx + 3) % 4
                if idx > 0:
                    wait_scatter(idx - 1, b3)
                pltpu.async_copy(hs.at[src_all.at[idx + 3]], msgs[b3],
                                 gsems[b3])

            def grp(i, _):
                for b in range(4):
                    idx = i * 4 + b
                    wait_gather(idx, b)
                    pltpu.async_copy(msgs[b], accum.at[dst_all.at[idx]],
                                     ssems[b], add=True)
                    b3 = (b + 3) % 4
                    wait_scatter(idx - 1, b3)
                    pltpu.async_copy(hs.at[src_all.at[idx + 3]], msgs[b3],
                                     gsems[b3])
                return _

            ngrp = (c - 3) // 4
            lax.fori_loop(1, ngrp, grp, 0)
            for idx in range(4 * ngrp, c):
                b = idx % 4
                wait_gather(idx, b)
                pltpu.async_copy(msgs[b], accum.at[dst_all.at[idx]], ssems[b],
                                 add=True)
                if idx + 3 < c:
                    b3 = (idx + 3) % 4
                    wait_scatter(idx - 1, b3)
                    pltpu.async_copy(hs.at[src_all.at[idx + 3]], msgs[b3],
                                     gsems[b3])
            for k in range(c - 4, c):
                wait_scatter(k, k % 4)

        plsc.subcore_barrier()
        for j in range(LKA):
            k = sid + j * NS
            if (j + 1) * NS <= NDA:
                pltpu.sync_copy(accum.at[pl.ds(k * EC, EC)], msg0)
                pltpu.sync_copy(msg0, out.at[cid, pl.ds(k * EC, EC)])
            else:
                @pl.when(k < NDA)
                def _():
                    pltpu.sync_copy(accum.at[pl.ds(k * EC, EC)], msg0)
                    pltpu.sync_copy(msg0, out.at[cid, pl.ds(k * EC, EC)])

    return pl.kernel(
        body,
        out_type=jax.ShapeDtypeStruct((NC, N, d), jnp.float32),
        mesh=_mesh(),
        scratch_types=[
            pltpu.VMEM((IB, EC), jnp.int32),
            pltpu.VMEM((IB, EC), jnp.int32),
            pltpu.VMEM((EC, d), jnp.float32),
            pltpu.VMEM((EC, d), jnp.float32),
            pltpu.VMEM((EC, d), jnp.float32),
            pltpu.VMEM((EC, d), jnp.float32),
            pltpu.VMEM_SHARED((N, d), jnp.float32),
            pltpu.SemaphoreType.DMA,
            pltpu.SemaphoreType.DMA,
            pltpu.SemaphoreType.DMA,
            pltpu.SemaphoreType.DMA,
            pltpu.SemaphoreType.DMA,
            pltpu.SemaphoreType.DMA,
            pltpu.SemaphoreType.DMA,
            pltpu.SemaphoreType.DMA,
        ],
    )


_agg128 = _make_agg(DH)


def _mm1_body(x_ref, w_ref, p0_ref, p1_ref, hs_ref, dinv_ref):
    deg = p0_ref[...] + p1_ref[...] + 1.0
    dinv = lax.rsqrt(deg)
    dinv_ref[...] = dinv
    h = jnp.dot(x_ref[...], w_ref[...], preferred_element_type=jnp.float32)
    hs_ref[...] = h * dinv[:, 0:1]


def _mm1_call(x, w, p0, p1):
    return pl.pallas_call(
        _mm1_body,
        grid=(GRID,),
        in_specs=[
            pl.BlockSpec((RB, DH), lambda i: (i, 0)),
            pl.BlockSpec((DH, DH), lambda i: (0, 0)),
            pl.BlockSpec((RB, 16), lambda i: (i, 0)),
            pl.BlockSpec((RB, 16), lambda i: (i, 0)),
        ],
        out_specs=[
            pl.BlockSpec((RB, DH), lambda i: (i, 0)),
            pl.BlockSpec((RB, 16), lambda i: (i, 0)),
        ],
        out_shape=[
            jax.ShapeDtypeStruct((N, DH), jnp.float32),
            jax.ShapeDtypeStruct((N, 16), jnp.float32),
        ],
    )(x, w, p0, p1)


def _comb_body(p0_ref, p1_ref, hs_ref, dinv_ref, b_ref, z_ref, s1_ref, s2_ref):
    i = pl.program_id(0)
    d = dinv_ref[...][:, 0:1]
    z = d * (p0_ref[...] + p1_ref[...] + hs_ref[...]) + b_ref[...]
    z_ref[...] = z
    s1 = jnp.sum(z, axis=0, keepdims=True)
    s2 = jnp.sum(z * z, axis=0, keepdims=True)

    @pl.when(i == 0)
    def _():
        s1_ref[...] = s1
        s2_ref[...] = s2

    @pl.when(i != 0)
    def _():
        s1_ref[...] += s1
        s2_ref[...] += s2


def _comb_call(p0, p1, hs, dinv, b):
    return pl.pallas_call(
        _comb_body,
        grid=(GRID,),
        in_specs=[
            pl.BlockSpec((RB, DH), lambda i: (i, 0)),
            pl.BlockSpec((RB, DH), lambda i: (i, 0)),
            pl.BlockSpec((RB, DH), lambda i: (i, 0)),
            pl.BlockSpec((RB, 16), lambda i: (i, 0)),
            pl.BlockSpec((1, DH), lambda i: (0, 0)),
        ],
        out_specs=[
            pl.BlockSpec((RB, DH), lambda i: (i, 0)),
            pl.BlockSpec((1, DH), lambda i: (0, 0)),
            pl.BlockSpec((1, DH), lambda i: (0, 0)),
        ],
        out_shape=[
            jax.ShapeDtypeStruct((N, DH), jnp.float32),
            jax.ShapeDtypeStruct((1, DH), jnp.float32),
            jax.ShapeDtypeStruct((1, DH), jnp.float32),
        ],
    )(p0, p1, hs, dinv, b)


def _bnmm_body(z_ref, s1_ref, s2_ref, g_ref, be_ref, dinv_ref, w_ref, out_ref):
    mu = s1_ref[...] * (1.0 / N)
    var = s2_ref[...] * (1.0 / N) - mu * mu
    rs = lax.rsqrt(var + 1e-5)
    a = jnp.maximum((z_ref[...] - mu) * rs * g_ref[...] + be_ref[...], 0.0)
    h = jnp.dot(a, w_ref[...], preferred_element_type=jnp.float32)
    out_ref[...] = h * dinv_ref[...][:, 0:1]


def _bnmm_call(z, s1, s2, g, be, dinv, w, dout):
    return pl.pallas_call(
        _bnmm_body,
        grid=(GRID,),
        in_specs=[
            pl.BlockSpec((RB, DH), lambda i: (i, 0)),
            pl.BlockSpec((1, DH), lambda i: (0, 0)),
            pl.BlockSpec((1, DH), lambda i: (0, 0)),
            pl.BlockSpec((1, DH), lambda i: (0, 0)),
            pl.BlockSpec((1, DH), lambda i: (0, 0)),
            pl.BlockSpec((RB, 16), lambda i: (i, 0)),
            pl.BlockSpec((DH, dout), lambda i: (0, 0)),
        ],
        out_specs=pl.BlockSpec((RB, dout), lambda i: (i, 0)),
        out_shape=jax.ShapeDtypeStruct((N, dout), jnp.float32),
    )(z, s1, s2, g, be, dinv, w)


def _fin_body(p0_ref, p1_ref, hs_ref, dinv_ref, b_ref, out_ref):
    d = dinv_ref[...][:, 0:1]
    z = d * (p0_ref[...] + p1_ref[...] + hs_ref[...]) + b_ref[...]
    col = lax.broadcasted_iota(jnp.int32, z.shape, 1)
    valid = col < DOUT
    zm = jnp.where(valid, z, -jnp.inf)
    m = jnp.max(zm, axis=1, keepdims=True)
    e = jnp.where(valid, jnp.exp(z - m), 0.0)
    lse = jnp.log(jnp.sum(e, axis=1, keepdims=True)) + m
    out_ref[...] = z - lse


def _fin_call(p0, p1, hs, dinv, b):
    return pl.pallas_call(
        _fin_body,
        grid=(GRID,),
        in_specs=[
            pl.BlockSpec((RB, D3), lambda i: (i, 0)),
            pl.BlockSpec((RB, D3), lambda i: (i, 0)),
            pl.BlockSpec((RB, D3), lambda i: (i, 0)),
            pl.BlockSpec((RB, 16), lambda i: (i, 0)),
            pl.BlockSpec((1, D3), lambda i: (0, 0)),
        ],
        out_specs=pl.BlockSpec((RB, D3), lambda i: (i, 0)),
        out_shape=jax.ShapeDtypeStruct((N, D3), jnp.float32),
    )(p0, p1, hs, dinv, b)


@jax.jit
def kernel(x, adj_t, W1, b1, g1, be1, W2, b2, g2, be2, W3, b3):
    adj3 = adj_t.reshape(2, NT, NCH, EC)
    zeros128 = jnp.zeros((EC, DH), jnp.float32)
    erows = jnp.zeros((EC, DH), jnp.float32).at[:, 0].set(1.0)

    degp = _deg_call(adj3, zeros128, erows)
    hs1, dinv16 = _mm1_call(x, W1, degp[0, :, :16], degp[1, :, :16])

    _agg = _agg128

    p1 = _agg(hs1, adj3, zeros128)
    z1, s11, s21 = _comb_call(p1[0], p1[1], hs1, dinv16, b1.reshape(1, DH))
    hs2 = _bnmm_call(z1, s11, s21, g1.reshape(1, DH), be1.reshape(1, DH),
                     dinv16, W2, DH)

    p2 = _agg(hs2, adj3, zeros128)
    z2, s12, s22 = _comb_call(p2[0], p2[1], hs2, dinv16, b2.reshape(1, DH))
    W3p = jnp.pad(W3, ((0, 0), (0, D3 - DOUT)))
    hs3 = _bnmm_call(z2, s12, s22, g2.reshape(1, DH), be2.reshape(1, DH),
                     dinv16, W3p, D3)

    p3 = _agg(hs3, adj3, zeros128)
    b3p = jnp.pad(b3, (0, D3 - DOUT)).reshape(1, D3)
    out64 = _fin_call(p3[0], p3[1], hs3, dinv16, b3p)
    return out64[:, :DOUT]


# speedup vs baseline: 1.0514x; 1.0514x over previous
import functools

import jax
import jax.numpy as jnp
from jax import lax
from jax.experimental import pallas as pl
from jax.experimental.pallas import tpu as pltpu
from jax.experimental.pallas import tpu_sc as plsc

N = 10000
E = 320000
DH = 128
DOUT = 40
D3 = 128

NC = 2
NS = 16
NT = NC * NS
EPT = E // NT
EC = 80
NCH = EPT // EC
RD = 200
NDCH = N // RD
LK = -(-NDCH // NS)
NDA = N // EC
LKA = -(-NDA // NS)
IB = 32
STAGES = (32, 32, 32, 29)

RB = 2000
GRID = N // RB


def _mesh():
    return plsc.VectorSubcoreMesh(core_axis_name="c", subcore_axis_name="s")


def _deg_body(adj3, zerosd, erows, out, dst_all, ones_v, zbuf, accum):
    cid = lax.axis_index("c")
    sid = lax.axis_index("s")
    tid = cid * NS + sid
    pltpu.sync_copy(erows, ones_v)
    pltpu.sync_copy(zerosd, zbuf)
    for j in range(LKA):
        k = sid + j * NS
        if (j + 1) * NS <= NDA:
            pltpu.sync_copy(zbuf, accum.at[pl.ds(k * EC, EC)])
        else:
            @pl.when(k < NDA)
            def _():
                pltpu.sync_copy(zbuf, accum.at[pl.ds(k * EC, EC)])
    pltpu.sync_copy(adj3.at[1, tid], dst_all)
    plsc.subcore_barrier()

    def body(i, c):
        pltpu.sync_copy(ones_v, accum.at[dst_all.at[i]], add=True)
        return c

    lax.fori_loop(0, NCH, body, 0)
    plsc.subcore_barrier()
    for j in range(LKA):
        k = sid + j * NS
        if (j + 1) * NS <= NDA:
            pltpu.sync_copy(accum.at[pl.ds(k * EC, EC)], zbuf)
            pltpu.sync_copy(zbuf, out.at[cid, pl.ds(k * EC, EC)])
        else:
            @pl.when(k < NDA)
            def _():
                pltpu.sync_copy(accum.at[pl.ds(k * EC, EC)], zbuf)
                pltpu.sync_copy(zbuf, out.at[cid, pl.ds(k * EC, EC)])


DG = DH

_deg_call = pl.kernel(
    _deg_body,
    out_type=jax.ShapeDtypeStruct((NC, N, DG), jnp.float32),
    mesh=_mesh(),
    scratch_types=[
        pltpu.VMEM((NCH, EC), jnp.int32),
        pltpu.VMEM((EC, DG), jnp.float32),
        pltpu.VMEM((EC, DG), jnp.float32),
        pltpu.VMEM_SHARED((N, DG), jnp.float32),
    ],
)


def _make_agg(d):
    def body(hs, adj3, zerosd, out, src_all, dst_all, msg0, msg1, msg2, msg3,
             accum, sem0, sem1, sem2, sem3):
        cid = lax.axis_index("c")
        sid = lax.axis_index("s")
        tid = cid * NS + sid
        pltpu.sync_copy(zerosd, msg0)
        for j in range(LKA):
            k = sid + j * NS
            if (j + 1) * NS <= NDA:
                pltpu.sync_copy(msg0, accum.at[pl.ds(k * EC, EC)])
            else:
                @pl.when(k < NDA)
                def _():
                    pltpu.sync_copy(msg0, accum.at[pl.ds(k * EC, EC)])
        plsc.subcore_barrier()

        msgs = (msg0, msg1, msg2, msg3)
        sems = (sem0, sem1, sem2, sem3)
        row = 0
        for c in STAGES:
            pltpu.sync_copy(adj3.at[0, tid, pl.ds(row, c)], src_all.at[pl.ds(0, c)])
            pltpu.sync_copy(adj3.at[1, tid, pl.ds(row, c)], dst_all.at[pl.ds(0, c)])
            row += c
            for p in range(3):
                pltpu.async_copy(hs.at[src_all.at[p]], msgs[p], sems[p])

            def grp(i, _):
                for b in range(4):
                    idx = i * 4 + b
                    pltpu.make_async_copy(hs.at[src_all.at[idx]], msgs[b],
                                          sems[b]).wait()
                    b2 = (b + 3) % 4
                    pltpu.async_copy(hs.at[src_all.at[idx + 3]], msgs[b2],
                                     sems[b2])
                    pltpu.sync_copy(msgs[b], accum.at[dst_all.at[idx]],
                                    add=True)
                return _

            ngrp = (c - 3) // 4
            lax.fori_loop(0, ngrp, grp, 0)
            for idx in range(4 * ngrp, c):
                b = idx % 4
                pltpu.make_async_copy(hs.at[src_all.at[idx]], msgs[b],
                                          sems[b]).wait()
                if idx + 3 < c:
                    b2 = (idx + 3) % 4
                    pltpu.async_copy(hs.at[src_all.at[idx + 3]], msgs[b2],
                                     sems[b2])
                pltpu.sync_copy(msgs[b], accum.at[dst_all.at[idx]], add=True)

        plsc.subcore_barrier()
        for j in range(LKA):
            k = sid + j * NS
            if (j + 1) * NS <= NDA:
                pltpu.sync_copy(accum.at[pl.ds(k * EC, EC)], msg0)
                pltpu.sync_copy(msg0, out.at[cid, pl.ds(k * EC, EC)])
            else:
                @pl.when(k < NDA)
                def _():
                    pltpu.sync_copy(accum.at[pl.ds(k * EC, EC)], msg0)
                    pltpu.sync_copy(msg0, out.at[cid, pl.ds(k * EC, EC)])

    return pl.kernel(
        body,
        out_type=jax.ShapeDtypeStruct((NC, N, d), jnp.float32),
        mesh=_mesh(),
        scratch_types=[
            pltpu.VMEM((IB, EC), jnp.int32),
            pltpu.VMEM((IB, EC), jnp.int32),
            pltpu.VMEM((EC, d), jnp.float32),
            pltpu.VMEM((EC, d), jnp.float32),
            pltpu.VMEM((EC, d), jnp.float32),
            pltpu.VMEM((EC, d), jnp.float32),
            pltpu.VMEM_SHARED((N, d), jnp.float32),
            pltpu.SemaphoreType.DMA,
            pltpu.SemaphoreType.DMA,
            pltpu.SemaphoreType.DMA,
            pltpu.SemaphoreType.DMA,
        ],
    )


_agg128 = _make_agg(DH)


def _mm1_body(x_ref, w_ref, p0_ref, p1_ref, hs_ref, dinv_ref):
    deg = p0_ref[...] + p1_ref[...] + 1.0
    dinv = lax.rsqrt(deg)
    dinv_ref[...] = dinv
    h = jnp.dot(x_ref[...], w_ref[...], preferred_element_type=jnp.float32)
    hs_ref[...] = h * dinv[:, 0:1]


def _mm1_call(x, w, p0, p1):
    return pl.pallas_call(
        _mm1_body,
        grid=(GRID,),
        in_specs=[
            pl.BlockSpec((RB, DH), lambda i: (i, 0)),
            pl.BlockSpec((DH, DH), lambda i: (0, 0)),
            pl.BlockSpec((RB, 16), lambda i: (i, 0)),
            pl.BlockSpec((RB, 16), lambda i: (i, 0)),
        ],
        out_specs=[
            pl.BlockSpec((RB, DH), lambda i: (i, 0)),
            pl.BlockSpec((RB, 16), lambda i: (i, 0)),
        ],
        out_shape=[
            jax.ShapeDtypeStruct((N, DH), jnp.float32),
            jax.ShapeDtypeStruct((N, 16), jnp.float32),
        ],
    )(x, w, p0, p1)


def _comb_body(p0_ref, p1_ref, hs_ref, dinv_ref, b_ref, z_ref, s1_ref, s2_ref):
    i = pl.program_id(0)
    d = dinv_ref[...][:, 0:1]
    z = d * (p0_ref[...] + p1_ref[...] + hs_ref[...]) + b_ref[...]
    z_ref[...] = z
    s1 = jnp.sum(z, axis=0, keepdims=True)
    s2 = jnp.sum(z * z, axis=0, keepdims=True)

    @pl.when(i == 0)
    def _():
        s1_ref[...] = s1
        s2_ref[...] = s2

    @pl.when(i != 0)
    def _():
        s1_ref[...] += s1
        s2_ref[...] += s2


def _comb_call(p0, p1, hs, dinv, b):
    return pl.pallas_call(
        _comb_body,
        grid=(GRID,),
        in_specs=[
            pl.BlockSpec((RB, DH), lambda i: (i, 0)),
            pl.BlockSpec((RB, DH), lambda i: (i, 0)),
            pl.BlockSpec((RB, DH), lambda i: (i, 0)),
            pl.BlockSpec((RB, 16), lambda i: (i, 0)),
            pl.BlockSpec((1, DH), lambda i: (0, 0)),
        ],
        out_specs=[
            pl.BlockSpec((RB, DH), lambda i: (i, 0)),
            pl.BlockSpec((1, DH), lambda i: (0, 0)),
            pl.BlockSpec((1, DH), lambda i: (0, 0)),
        ],
        out_shape=[
            jax.ShapeDtypeStruct((N, DH), jnp.float32),
            jax.ShapeDtypeStruct((1, DH), jnp.float32),
            jax.ShapeDtypeStruct((1, DH), jnp.float32),
        ],
    )(p0, p1, hs, dinv, b)


def _bnmm_body(z_ref, s1_ref, s2_ref, g_ref, be_ref, dinv_ref, w_ref, out_ref):
    mu = s1_ref[...] * (1.0 / N)
    var = s2_ref[...] * (1.0 / N) - mu * mu
    rs = lax.rsqrt(var + 1e-5)
    a = jnp.maximum((z_ref[...] - mu) * rs * g_ref[...] + be_ref[...], 0.0)
    h = jnp.dot(a, w_ref[...], preferred_element_type=jnp.float32)
    out_ref[...] = h * dinv_ref[...][:, 0:1]


def _bnmm_call(z, s1, s2, g, be, dinv, w, dout):
    return pl.pallas_call(
        _bnmm_body,
        grid=(GRID,),
        in_specs=[
            pl.BlockSpec((RB, DH), lambda i: (i, 0)),
            pl.BlockSpec((1, DH), lambda i: (0, 0)),
            pl.BlockSpec((1, DH), lambda i: (0, 0)),
            pl.BlockSpec((1, DH), lambda i: (0, 0)),
            pl.BlockSpec((1, DH), lambda i: (0, 0)),
            pl.BlockSpec((RB, 16), lambda i: (i, 0)),
            pl.BlockSpec((DH, dout), lambda i: (0, 0)),
        ],
        out_specs=pl.BlockSpec((RB, dout), lambda i: (i, 0)),
        out_shape=jax.ShapeDtypeStruct((N, dout), jnp.float32),
    )(z, s1, s2, g, be, dinv, w)


def _fin_body(p0_ref, p1_ref, hs_ref, dinv_ref, b_ref, out_ref):
    d = dinv_ref[...][:, 0:1]
    z = d * (p0_ref[...] + p1_ref[...] + hs_ref[...]) + b_ref[...]
    col = lax.broadcasted_iota(jnp.int32, z.shape, 1)
    valid = col < DOUT
    zm = jnp.where(valid, z, -jnp.inf)
    m = jnp.max(zm, axis=1, keepdims=True)
    e = jnp.where(valid, jnp.exp(z - m), 0.0)
    lse = jnp.log(jnp.sum(e, axis=1, keepdims=True)) + m
    out_ref[...] = z - lse


def _fin_call(p0, p1, hs, dinv, b):
    return pl.pallas_call(
        _fin_body,
        grid=(GRID,),
        in_specs=[
            pl.BlockSpec((RB, D3), lambda i: (i, 0)),
            pl.BlockSpec((RB, D3), lambda i: (i, 0)),
            pl.BlockSpec((RB, D3), lambda i: (i, 0)),
            pl.BlockSpec((RB, 16), lambda i: (i, 0)),
            pl.BlockSpec((1, D3), lambda i: (0, 0)),
        ],
        out_specs=pl.BlockSpec((RB, D3), lambda i: (i, 0)),
        out_shape=jax.ShapeDtypeStruct((N, D3), jnp.float32),
    )(p0, p1, hs, dinv, b)


@jax.jit
def kernel(x, adj_t, W1, b1, g1, be1, W2, b2, g2, be2, W3, b3):
    adj3 = adj_t.reshape(2, NT, NCH, EC)
    zeros128 = jnp.zeros((EC, DH), jnp.float32)
    zerosdg = jnp.zeros((EC, DG), jnp.float32)
    erows = jnp.zeros((EC, DG), jnp.float32).at[:, 0].set(1.0)

    degp = _deg_call(adj3, zerosdg, erows)
    hs1, dinv16 = _mm1_call(x, W1, degp[0, :, :16], degp[1, :, :16])

    _agg = _agg128

    p1 = _agg(hs1, adj3, zeros128)
    z1, s11, s21 = _comb_call(p1[0], p1[1], hs1, dinv16, b1.reshape(1, DH))
    hs2 = _bnmm_call(z1, s11, s21, g1.reshape(1, DH), be1.reshape(1, DH),
                     dinv16, W2, DH)

    p2 = _agg(hs2, adj3, zeros128)
    z2, s12, s22 = _comb_call(p2[0], p2[1], hs2, dinv16, b2.reshape(1, DH))
    W3p = jnp.pad(W3, ((0, 0), (0, D3 - DOUT)))
    hs3 = _bnmm_call(z2, s12, s22, g2.reshape(1, DH), be2.reshape(1, DH),
                     dinv16, W3p, D3)

    p3 = _agg(hs3, adj3, zeros128)
    b3p = jnp.pad(b3, (0, D3 - DOUT)).reshape(1, D3)
    out64 = _fin_call(p3[0], p3[1], hs3, dinv16, b3p)
    return out64[:, :DOUT]
